# Initial kernel scaffold; baseline (speedup 1.0000x reference)
import jax
import jax.numpy as jnp
from jax.experimental import pallas as pl


def kernel(point_cloud):
    raise NotImplementedError("write your pallas kernel here")


# speedup vs baseline: 1.4507x; 1.4507x over previous
import functools

import numpy as np
import jax
import jax.numpy as jnp
from jax import lax
from jax.experimental import pallas as pl
from jax.experimental.pallas import tpu as pltpu
from jax.experimental.pallas import tpu_sc as plsc

R = 256
EPS = 1e-10
N = 2000000
W = 750
PPR = W // 3
ROWS = (N * 3) // W
BLKR = 400
GRID = R * R * R
IROWS = N // 128

NSUB = 16
CHR = 16
ROWS_PER_TILE = 976
NCHUNK = ROWS_PER_TILE // CHR
LEFTOVER_BASE = NSUB * ROWS_PER_TILE
ZWORDS = 65536
ZCOPIES = GRID // NSUB // ZWORDS


def _minmax_body(x_ref, min_ref, max_ref):
    i = pl.program_id(0)
    x = x_ref[...]
    pmin = jnp.min(x, axis=0, keepdims=True)
    pmax = jnp.max(x, axis=0, keepdims=True)

    @pl.when(i == 0)
    def _():
        min_ref[...] = pmin
        max_ref[...] = pmax

    @pl.when(i != 0)
    def _():
        min_ref[...] = jnp.minimum(min_ref[...], pmin)
        max_ref[...] = jnp.maximum(max_ref[...], pmax)


def _quant_body(x_ref, minv_ref, scalev_ref, s_ref, out_ref):
    t = (x_ref[...] - minv_ref[...]) / scalev_ref[...] * jnp.float32(R - 1)
    q = jnp.floor(jnp.clip(t, 0.0, jnp.float32(R - 1)))
    lin = lax.dot_general(q, s_ref[...], (((1,), (0,)), ((), ())),
                          preferred_element_type=jnp.float32)
    out_ref[...] = lin.astype(jnp.int32)


def _tobool_body(g_ref, out_ref):
    out_ref[...] = g_ref[...] != 0


def _sc_scatter_body(idx_hbm, grid_hbm, zbuf, idx_buf, extra_buf, ones_ref,
                     zsem, ssem):
    core = lax.axis_index("c")
    tile = lax.axis_index("s")

    @pl.when(core == 0)
    def _():
        def zinit(i, carry):
            zbuf[pl.ds(i * 16, 16)] = jnp.zeros((16,), jnp.int32)
            return carry
        lax.fori_loop(0, ZWORDS // 16, zinit, 0)
        for k in range(8):
            ones_ref[pl.ds(k * 16, 16)] = jnp.ones((16,), jnp.int32)

        zdescs = []
        for k in range(ZCOPIES):
            off = tile * (GRID // NSUB) + k * ZWORDS
            zdescs.append(pltpu.async_copy(
                zbuf, grid_hbm.at[pl.ds(off, ZWORDS)], zsem))
        for d in zdescs:
            d.wait()

    plsc.subcore_barrier()

    @pl.when(core == 0)
    def _():
        start = tile * ROWS_PER_TILE

        def chunk(c, carry):
            pltpu.sync_copy(idx_hbm.at[pl.ds(start + c * CHR, CHR)], idx_buf)
            descs = []
            for j in range(CHR):
                descs.append(pltpu.async_copy(
                    ones_ref, grid_hbm.at[idx_buf.at[j]], ssem))
            for d in descs:
                d.wait()
            return carry
        lax.fori_loop(0, NCHUNK, chunk, 0)

        @pl.when(tile < IROWS - LEFTOVER_BASE)
        def _():
            pltpu.sync_copy(idx_hbm.at[pl.ds(LEFTOVER_BASE + tile, 1)],
                            extra_buf)
            pltpu.async_copy(ones_ref, grid_hbm.at[extra_buf.at[0]],
                             ssem).wait()


def _make_s_matrix():
    s = np.zeros((W, PPR), np.float32)
    w = np.array([R * R, R, 1], np.float32)
    for j in range(W):
        s[j, j // 3] = w[j % 3]
    return jnp.asarray(s)


_sc_scatter = functools.partial(
    pl.kernel,
    out_type=jax.ShapeDtypeStruct((GRID,), jnp.int32),
    mesh=plsc.VectorSubcoreMesh(core_axis_name="c", subcore_axis_name="s"),
    scratch_types=[
        pltpu.VMEM((ZWORDS,), jnp.int32),
        pltpu.VMEM((CHR, 128), jnp.int32),
        pltpu.VMEM((1, 128), jnp.int32),
        pltpu.VMEM((128,), jnp.int32),
        pltpu.SemaphoreType.DMA,
        pltpu.SemaphoreType.DMA,
    ],
)(_sc_scatter_body)


@jax.jit
def kernel(point_cloud):
    pts = point_cloud.reshape(ROWS, W)

    minv, maxv = pl.pallas_call(
        _minmax_body,
        grid=(ROWS // BLKR,),
        in_specs=[pl.BlockSpec((BLKR, W), lambda i: (i, 0))],
        out_specs=[pl.BlockSpec((1, W), lambda i: (0, 0)),
                   pl.BlockSpec((1, W), lambda i: (0, 0))],
        out_shape=[jax.ShapeDtypeStruct((1, W), jnp.float32),
                   jax.ShapeDtypeStruct((1, W), jnp.float32)],
    )(pts)

    min_bounds = jnp.min(minv[0].reshape(PPR, 3), axis=0)
    max_bounds = jnp.max(maxv[0].reshape(PPR, 3), axis=0)
    scale = max_bounds - min_bounds
    scale = jnp.where(scale == 0, jnp.ones_like(scale) * jnp.float32(EPS),
                      scale)

    minvw = jnp.tile(min_bounds, PPR).reshape(1, W)
    scalevw = jnp.tile(scale, PPR).reshape(1, W)

    idx = pl.pallas_call(
        _quant_body,
        grid=(ROWS // BLKR,),
        in_specs=[pl.BlockSpec((BLKR, W), lambda i: (i, 0)),
                  pl.BlockSpec((1, W), lambda i: (0, 0)),
                  pl.BlockSpec((1, W), lambda i: (0, 0)),
                  pl.BlockSpec((W, PPR), lambda i: (0, 0))],
        out_specs=pl.BlockSpec((BLKR, PPR), lambda i: (i, 0)),
        out_shape=jax.ShapeDtypeStruct((ROWS, PPR), jnp.int32),
    )(pts, minvw, scalevw, _make_s_matrix())

    grid_i32 = _sc_scatter(idx.reshape(IROWS, 128))

    grid_bool = pl.pallas_call(
        _tobool_body,
        grid=(8,),
        in_specs=[pl.BlockSpec((512, 4096), lambda i: (i, 0))],
        out_specs=pl.BlockSpec((512, 4096), lambda i: (i, 0)),
        out_shape=jax.ShapeDtypeStruct((4096, 4096), jnp.bool_),
    )(grid_i32.reshape(4096, 4096))

    return (grid_bool.reshape(R, R, R), min_bounds, max_bounds, scale)


# speedup vs baseline: 5.1897x; 3.5773x over previous
import functools

import jax
import jax.numpy as jnp
from jax import lax
from jax.experimental import pallas as pl
from jax.experimental.pallas import tpu as pltpu
from jax.experimental.pallas import tpu_sc as plsc

R = 256
EPS = 1e-10
N = 2000000
CROWS = 16000
CCOLS = 125
BLKR = 1000
GRID = R * R * R

NSUB = 16
ROWS_PER_TILE = CROWS // NSUB
CHR = 40
NCHUNK = ROWS_PER_TILE // CHR
ZWORDS = 65536
ZCOPIES = GRID // NSUB // ZWORDS


def _minmax_body(x_ref, y_ref, z_ref, xn_ref, xx_ref, yn_ref, yx_ref,
                 zn_ref, zx_ref):
    i = pl.program_id(0)
    x, y, z = x_ref[...], y_ref[...], z_ref[...]

    @pl.when(i == 0)
    def _():
        xn_ref[0] = jnp.min(x)
        xx_ref[0] = jnp.max(x)
        yn_ref[0] = jnp.min(y)
        yx_ref[0] = jnp.max(y)
        zn_ref[0] = jnp.min(z)
        zx_ref[0] = jnp.max(z)

    @pl.when(i != 0)
    def _():
        xn_ref[0] = jnp.minimum(xn_ref[0], jnp.min(x))
        xx_ref[0] = jnp.maximum(xx_ref[0], jnp.max(x))
        yn_ref[0] = jnp.minimum(yn_ref[0], jnp.min(y))
        yx_ref[0] = jnp.maximum(yx_ref[0], jnp.max(y))
        zn_ref[0] = jnp.minimum(zn_ref[0], jnp.min(z))
        zx_ref[0] = jnp.maximum(zx_ref[0], jnp.max(z))


def _quant_body(mins_ref, scales_ref, x_ref, y_ref, z_ref, out_ref):
    r = jnp.float32(R - 1)
    qx = jnp.floor(jnp.clip((x_ref[...] - mins_ref[0]) / scales_ref[0] * r,
                            0.0, r)).astype(jnp.int32)
    qy = jnp.floor(jnp.clip((y_ref[...] - mins_ref[1]) / scales_ref[1] * r,
                            0.0, r)).astype(jnp.int32)
    qz = jnp.floor(jnp.clip((z_ref[...] - mins_ref[2]) / scales_ref[2] * r,
                            0.0, r)).astype(jnp.int32)
    out_ref[...] = (qx << 16) | (qy << 8) | qz


def _tobool_body(g_ref, out_ref):
    out_ref[...] = g_ref[...] != 0


def _sc_scatter_body(idx_hbm, grid_hbm, zbuf, idx_buf, ones_ref, zsem, ssem):
    core = lax.axis_index("c")
    tile = lax.axis_index("s")

    @pl.when(core == 0)
    def _():
        def zinit(i, carry):
            zbuf[pl.ds(i * 16, 16)] = jnp.zeros((16,), jnp.int32)
            return carry
        lax.fori_loop(0, ZWORDS // 16, zinit, 0)
        for k in range(7):
            ones_ref[pl.ds(k * 16, 16)] = jnp.ones((16,), jnp.int32)
        ones_ref[pl.ds(CCOLS - 16, 16)] = jnp.ones((16,), jnp.int32)

        zdescs = []
        for k in range(ZCOPIES):
            off = tile * (GRID // NSUB) + k * ZWORDS
            zdescs.append(pltpu.async_copy(
                zbuf, grid_hbm.at[pl.ds(off, ZWORDS)], zsem))
        for d in zdescs:
            d.wait()

    plsc.subcore_barrier()

    @pl.when(core == 0)
    def _():
        start = tile * ROWS_PER_TILE

        def chunk(c, carry):
            pltpu.sync_copy(idx_hbm.at[pl.ds(start + c * CHR, CHR)], idx_buf)
            descs = []
            for j in range(CHR):
                descs.append(pltpu.async_copy(
                    ones_ref, grid_hbm.at[idx_buf.at[j]], ssem))
            for d in descs:
                d.wait()
            return carry
        lax.fori_loop(0, NCHUNK, chunk, 0)


_sc_scatter = functools.partial(
    pl.kernel,
    out_type=jax.ShapeDtypeStruct((GRID,), jnp.int32),
    mesh=plsc.VectorSubcoreMesh(core_axis_name="c", subcore_axis_name="s"),
    scratch_types=[
        pltpu.VMEM((ZWORDS,), jnp.int32),
        pltpu.VMEM((CHR, CCOLS), jnp.int32),
        pltpu.VMEM((CCOLS,), jnp.int32),
        pltpu.SemaphoreType.DMA,
        pltpu.SemaphoreType.DMA,
    ],
)(_sc_scatter_body)


@jax.jit
def kernel(point_cloud):
    xs = point_cloud[:, 0].reshape(CROWS, CCOLS)
    ys = point_cloud[:, 1].reshape(CROWS, CCOLS)
    zs = point_cloud[:, 2].reshape(CROWS, CCOLS)

    blk = pl.BlockSpec((BLKR, CCOLS), lambda i: (i, 0))
    sout = pl.BlockSpec(memory_space=pltpu.SMEM)
    s1 = jax.ShapeDtypeStruct((1,), jnp.float32)

    xn, xx, yn, yx, zn, zx = pl.pallas_call(
        _minmax_body,
        grid=(CROWS // BLKR,),
        in_specs=[blk, blk, blk],
        out_specs=[sout] * 6,
        out_shape=[s1] * 6,
    )(xs, ys, zs)

    min_bounds = jnp.concatenate([xn, yn, zn])
    max_bounds = jnp.concatenate([xx, yx, zx])
    scale = max_bounds - min_bounds
    scale = jnp.where(scale == 0, jnp.ones_like(scale) * jnp.float32(EPS),
                      scale)

    idx = pl.pallas_call(
        _quant_body,
        grid=(CROWS // BLKR,),
        in_specs=[pl.BlockSpec(memory_space=pltpu.SMEM),
                  pl.BlockSpec(memory_space=pltpu.SMEM),
                  blk, blk, blk],
        out_specs=blk,
        out_shape=jax.ShapeDtypeStruct((CROWS, CCOLS), jnp.int32),
    )(min_bounds, scale, xs, ys, zs)

    grid_i32 = _sc_scatter(idx)

    grid_bool = pl.pallas_call(
        _tobool_body,
        grid=(8,),
        in_specs=[pl.BlockSpec((512, 4096), lambda i: (i, 0))],
        out_specs=pl.BlockSpec((512, 4096), lambda i: (i, 0)),
        out_shape=jax.ShapeDtypeStruct((4096, 4096), jnp.bool_),
    )(grid_i32.reshape(4096, 4096))

    return (grid_bool.reshape(R, R, R), min_bounds, max_bounds, scale)
